# baseline (speedup 1.0000x reference)
import functools

import jax
import jax.numpy as jnp
from jax.experimental import pallas as pl
import jax.experimental.pallas.tpu as pltpu


def _blk_kernel(idx_ref, w_ref, x_ref, b_ref, out_ref, s_ref, *,
                in_features, out_features):
    fan = idx_ref.shape[0]
    bo = 256

    @pl.when(pl.program_id(0) == 0)
    def _build_s():
        iota = jax.lax.broadcasted_iota(jnp.int16, (in_features, bo), 0)
        for blk in range(out_features // bo):
            idx = idx_ref[:, blk * bo : (blk + 1) * bo]
            w = w_ref[:, blk * bo : (blk + 1) * bo]
            s = jnp.zeros((in_features, bo), jnp.bfloat16)
            for f in range(fan):
                s = jnp.where(iota == idx[f : f + 1, :], w[f : f + 1, :], s)
            s_ref[:, blk * bo : (blk + 1) * bo] = s

    out_ref[...] = (
        jnp.dot(
            x_ref[...].astype(jnp.bfloat16),
            s_ref[...],
            preferred_element_type=jnp.float32,
        )
        + b_ref[...]
    )


def kernel(input, weight, bias, indx_seqs):
    batch, in_features = input.shape
    out_features, fan_in = weight.shape
    bb = min(1024, batch)
    n_blk = batch // bb

    idx_t = indx_seqs.astype(jnp.int16).T
    w_t = weight.T.astype(jnp.bfloat16)
    bias2 = bias.reshape(1, out_features)

    out = pl.pallas_call(
        functools.partial(
            _blk_kernel, in_features=in_features, out_features=out_features
        ),
        grid=(n_blk,),
        in_specs=[
            pl.BlockSpec((fan_in, out_features), lambda j: (0, 0)),
            pl.BlockSpec((fan_in, out_features), lambda j: (0, 0)),
            pl.BlockSpec((bb, in_features), lambda j: (j, 0)),
            pl.BlockSpec((1, out_features), lambda j: (0, 0)),
        ],
        out_specs=pl.BlockSpec((bb, out_features), lambda j: (j, 0)),
        out_shape=jax.ShapeDtypeStruct((batch, out_features), jnp.float32),
        scratch_shapes=[
            pltpu.VMEM((in_features, out_features), jnp.bfloat16),
        ],
    )(idx_t, w_t, input, bias2)
    return out
